# baseline (speedup 1.0000x reference)
import functools

import jax
import jax.numpy as jnp
from jax import lax
from jax.experimental import pallas as pl
from jax.experimental.pallas import tpu as pltpu
from jax.experimental.pallas import tpu_sc as plsc

B = 16
T = 1024
D = 256
MAX_LEN = 4096
NW = 32
FRAMES_PER_W = B * MAX_LEN // NW
CHUNK = 128
NCHUNK = FRAMES_PER_W // CHUNK
NBUF = 3
ZROWS = 64
MAX_DUR = 7

_mesh = plsc.VectorSubcoreMesh(core_axis_name="c", subcore_axis_name="s")


@functools.partial(
    pl.kernel,
    mesh=_mesh,
    compiler_params=pltpu.CompilerParams(needs_layout_passes=False, use_tc_tiling_on_sc=False),
    out_type=[
        jax.ShapeDtypeStruct((B * MAX_LEN, D), jnp.float32),
        jax.ShapeDtypeStruct((B, 16), jnp.int32),
    ],
    scratch_types=[
        pltpu.VMEM((T,), jnp.int32),
        pltpu.VMEM((NCHUNK, CHUNK), jnp.int32),
        pltpu.VMEM((NBUF, CHUNK, D), jnp.float32),
        pltpu.VMEM((ZROWS, D), jnp.float32),
        pltpu.VMEM((16,), jnp.int32),
        pltpu.SemaphoreType.DMA((NBUF,)),
        pltpu.SemaphoreType.DMA((NBUF,)),
        pltpu.SemaphoreType.DMA,
    ],
)
def _regulate(x_hbm, dur_hbm, out_hbm, mel_hbm, dur_v, idx_v, rows_v, zbuf,
              mel_v, gsem, ssem, zsem):
    cid = lax.axis_index("c")
    sid = lax.axis_index("s")
    b = sid
    half = (sid + cid) % 2
    wid = sid * 2 + cid
    obase = b * MAX_LEN + half * CHUNK

    pltpu.sync_copy(dur_hbm.at[pl.ds(b * T, T)], dur_v)

    lanes = jnp.arange(16, dtype=jnp.int32)
    zeros16f = jnp.zeros((16,), jnp.float32)

    idx_init = lanes * T + wid * 32

    def init_body(i, carry):
        idx_v[i // (CHUNK // 16), pl.ds((i % (CHUNK // 16)) * 16, 16)] = (
            idx_init)
        return carry

    lax.fori_loop(0, FRAMES_PER_W // 16, init_body, 0)

    def ph_body(j, carry):
        d = dur_v[pl.ds(j * 16, 16)]
        c_inc = plsc.cumsum(d) + carry
        start = c_inc - d
        gvec = b * T + j * 16 + lanes
        for k in range(MAX_DUR):
            p = start + k
            gc = p >> 7
            m = (d > k) & (p < MAX_LEN) & ((gc & 1) == half)
            lp = ((gc >> 1) << 7) | (p & (CHUNK - 1))
            lpc = jnp.minimum(lp, FRAMES_PER_W - 1)
            plsc.store_scatter(idx_v, [lpc >> 7, lpc & (CHUNK - 1)], gvec,
                               mask=m)
        return carry + jnp.sum(d)

    total = lax.fori_loop(0, T // 16, ph_body, jnp.int32(0))

    vl = jnp.minimum(total, MAX_LEN)
    gvc = (vl + CHUNK - 1) // CHUNK
    nvc = (gvc - half + 1) // 2
    nvcm1 = jnp.maximum(nvc - 1, 0)
    rounds = jnp.maximum((nvc + NBUF - 1) // NBUF - 1, 0)

    def g_issue(slot, ci):
        return pltpu.async_copy(
            x_hbm.at[idx_v.at[ci]], rows_v.at[slot], gsem.at[slot])

    def g_wait(slot):
        pltpu.make_async_copy(
            x_hbm.at[idx_v.at[0]], rows_v.at[slot], gsem.at[slot]).wait()

    def s_issue(slot, ci):
        return pltpu.async_copy(
            rows_v.at[slot],
            out_hbm.at[pl.ds(obase + ci * (2 * CHUNK), CHUNK)],
            ssem.at[slot])

    def s_wait(slot):
        pltpu.make_async_copy(
            rows_v.at[slot],
            out_hbm.at[pl.ds(obase, CHUNK)],
            ssem.at[slot]).wait()

    def fixup(slot, ci):
        lo = jnp.clip(vl - (2 * ci + half) * CHUNK, 0, CHUNK)

        def fb(r, carry):
            for k in range(D // 16):
                rows_v[slot, r, pl.ds(k * 16, 16)] = zeros16f
            return carry

        lax.fori_loop(lo, CHUNK, fb, 0)

    for slot in range(NBUF):
        g_issue(slot, jnp.minimum(jnp.int32(slot), nvcm1))

    def zinit(i, carry):
        zbuf[i // (D // 16), pl.ds((i % (D // 16)) * 16, 16)] = zeros16f
        return carry

    lax.fori_loop(0, ZROWS * D // 16, zinit, 0)

    def z_issue(zc, carry):
        for q in range(CHUNK // ZROWS):
            pltpu.async_copy(
                zbuf,
                out_hbm.at[pl.ds(obase + zc * (2 * CHUNK) + q * ZROWS,
                                 ZROWS)],
                zsem)
        return carry

    lax.fori_loop(nvc, NCHUNK, z_issue, 0)

    mel_v[...] = jnp.full((16,), total, jnp.int32)

    @pl.when(half == 0)
    def _():
        pltpu.sync_copy(mel_v, mel_hbm.at[b])

    def ring_body(r, carry):
        for slot in range(NBUF):
            g_wait(slot)
            ci = jnp.minimum(r * NBUF + slot, nvcm1)
            fixup(slot, ci)
            s_issue(slot, ci)
        for slot in range(NBUF):
            s_wait(slot)
            g_issue(slot, jnp.minimum((r + 1) * NBUF + slot, nvcm1))
        return carry

    lax.fori_loop(0, rounds, ring_body, 0)

    for slot in range(NBUF):
        g_wait(slot)
        ci = jnp.minimum(rounds * NBUF + slot, nvcm1)
        fixup(slot, ci)
        s_issue(slot, ci)
    for slot in range(NBUF):
        s_wait(slot)

    def z_wait(zc, carry):
        for q in range(CHUNK // ZROWS):
            pltpu.make_async_copy(
                zbuf,
                out_hbm.at[pl.ds(obase + zc * (2 * CHUNK) + q * ZROWS,
                                 ZROWS)],
                zsem).wait()
        return carry

    lax.fori_loop(nvc, NCHUNK, z_wait, 0)


def kernel(x, duration, max_len):
    del max_len
    out_flat, mel2d = _regulate(
        x.reshape(B * T, D), duration.reshape(-1).astype(jnp.int32))
    return out_flat.reshape(B, MAX_LEN, D), mel2d[:, 0]


# speedup vs baseline: 2.0931x; 2.0931x over previous
import functools

import jax
import jax.numpy as jnp
from jax import lax
from jax.experimental import pallas as pl
from jax.experimental.pallas import tpu as pltpu
from jax.experimental.pallas import tpu_sc as plsc

B = 16
T = 1024
D = 256
MAX_LEN = 4096
NW = 32
FRAMES_PER_W = B * MAX_LEN // NW
CHUNK = 128
NCHUNK = FRAMES_PER_W // CHUNK
NBUF = 3
ZROWS = 64
MAX_DUR = 7

_mesh = plsc.VectorSubcoreMesh(core_axis_name="c", subcore_axis_name="s")


@functools.partial(
    pl.kernel,
    mesh=_mesh,
    compiler_params=pltpu.CompilerParams(needs_layout_passes=False),
    out_type=[
        jax.ShapeDtypeStruct((B * MAX_LEN, D), jnp.float32),
        jax.ShapeDtypeStruct((B, 16), jnp.int32),
    ],
    scratch_types=[
        pltpu.VMEM((T,), jnp.int32),
        pltpu.VMEM((NCHUNK, CHUNK), jnp.int32),
        pltpu.VMEM((NBUF, CHUNK, D), jnp.float32),
        pltpu.VMEM((ZROWS, D), jnp.float32),
        pltpu.VMEM((16,), jnp.int32),
        pltpu.SemaphoreType.DMA((NBUF,)),
        pltpu.SemaphoreType.DMA((NBUF,)),
        pltpu.SemaphoreType.DMA,
    ],
)
def _regulate(x_hbm, dur_hbm, out_hbm, mel_hbm, dur_v, idx_v, rows_v, zbuf,
              mel_v, gsem, ssem, zsem):
    cid = lax.axis_index("c")
    sid = lax.axis_index("s")
    b = sid
    half = (sid + cid) % 2
    wid = sid * 2 + cid
    obase = b * MAX_LEN + half * CHUNK

    pltpu.sync_copy(dur_hbm.at[pl.ds(b * T, T)], dur_v)

    lanes = jnp.arange(16, dtype=jnp.int32)
    zeros16f = jnp.zeros((16,), jnp.float32)

    idx_init = lanes * T + wid * 32

    def init_body(i, carry):
        idx_v[i // (CHUNK // 16), pl.ds((i % (CHUNK // 16)) * 16, 16)] = (
            idx_init)
        return carry

    lax.fori_loop(0, FRAMES_PER_W // 16, init_body, 0)

    def ph_body(j, carry):
        d = dur_v[pl.ds(j * 16, 16)]
        c_inc = plsc.cumsum(d) + carry
        start = c_inc - d
        gvec = b * T + j * 16 + lanes
        for k in range(MAX_DUR):
            p = start + k
            gc = p >> 7
            m = (d > k) & (p < MAX_LEN) & ((gc & 1) == half)
            lp = ((gc >> 1) << 7) | (p & (CHUNK - 1))
            lpc = jnp.minimum(lp, FRAMES_PER_W - 1)
            plsc.store_scatter(idx_v, [lpc >> 7, lpc & (CHUNK - 1)], gvec,
                               mask=m)
        return carry + jnp.sum(d)

    total = lax.fori_loop(0, T // 16, ph_body, jnp.int32(0))

    vl = jnp.minimum(total, MAX_LEN)
    gvc = (vl + CHUNK - 1) // CHUNK
    nvc = (gvc - half + 1) // 2
    nvcm1 = jnp.maximum(nvc - 1, 0)
    rounds = jnp.maximum((nvc + NBUF - 1) // NBUF - 1, 0)

    def g_issue(slot, ci):
        return pltpu.async_copy(
            x_hbm.at[idx_v.at[ci]], rows_v.at[slot], gsem.at[slot])

    def g_wait(slot):
        pltpu.make_async_copy(
            x_hbm.at[idx_v.at[0]], rows_v.at[slot], gsem.at[slot]).wait()

    def s_issue(slot, ci):
        return pltpu.async_copy(
            rows_v.at[slot],
            out_hbm.at[pl.ds(obase + ci * (2 * CHUNK), CHUNK)],
            ssem.at[slot])

    def s_wait(slot):
        pltpu.make_async_copy(
            rows_v.at[slot],
            out_hbm.at[pl.ds(obase, CHUNK)],
            ssem.at[slot]).wait()

    def fixup(slot, ci):
        lo = jnp.clip(vl - (2 * ci + half) * CHUNK, 0, CHUNK)

        def fb(r, carry):
            for k in range(D // 16):
                rows_v[slot, r, pl.ds(k * 16, 16)] = zeros16f
            return carry

        lax.fori_loop(lo, CHUNK, fb, 0)

    for slot in range(NBUF):
        g_issue(slot, jnp.minimum(jnp.int32(slot), nvcm1))

    def zinit(i, carry):
        zbuf[i // (D // 16), pl.ds((i % (D // 16)) * 16, 16)] = zeros16f
        return carry

    lax.fori_loop(0, ZROWS * D // 16, zinit, 0)

    def z_issue(zc, carry):
        for q in range(CHUNK // ZROWS):
            pltpu.async_copy(
                zbuf,
                out_hbm.at[pl.ds(obase + zc * (2 * CHUNK) + q * ZROWS,
                                 ZROWS)],
                zsem)
        return carry

    lax.fori_loop(nvc, NCHUNK, z_issue, 0)

    mel_v[...] = jnp.full((16,), total, jnp.int32)

    @pl.when(half == 0)
    def _():
        pltpu.sync_copy(mel_v, mel_hbm.at[b])

    def ring_body(r, carry):
        for slot in range(NBUF):
            g_wait(slot)
            ci = jnp.minimum(r * NBUF + slot, nvcm1)
            fixup(slot, ci)
            s_issue(slot, ci)
        for slot in range(NBUF):
            s_wait(slot)
            g_issue(slot, jnp.minimum((r + 1) * NBUF + slot, nvcm1))
        return carry

    lax.fori_loop(0, rounds, ring_body, 0)

    for slot in range(NBUF):
        g_wait(slot)
        ci = jnp.minimum(rounds * NBUF + slot, nvcm1)
        fixup(slot, ci)
        s_issue(slot, ci)
    for slot in range(NBUF):
        s_wait(slot)

    def z_wait(zc, carry):
        for q in range(CHUNK // ZROWS):
            pltpu.make_async_copy(
                zbuf,
                out_hbm.at[pl.ds(obase + zc * (2 * CHUNK) + q * ZROWS,
                                 ZROWS)],
                zsem).wait()
        return carry

    lax.fori_loop(nvc, NCHUNK, z_wait, 0)


def kernel(x, duration, max_len):
    del max_len
    out_flat, mel2d = _regulate(
        x.reshape(B * T, D), duration.reshape(-1).astype(jnp.int32))
    return out_flat.reshape(B, MAX_LEN, D), mel2d[:, 0]
